# Initial kernel scaffold; baseline (speedup 1.0000x reference)
import jax
import jax.numpy as jnp
from jax.experimental import pallas as pl


def kernel(features, W1, b1, W2, b2):
    raise NotImplementedError("write your pallas kernel here")


# speedup vs baseline: 13.9254x; 13.9254x over previous
import functools

import jax
import jax.numpy as jnp
from jax import lax
from jax.experimental import pallas as pl

K = 30


def _emb_body(x_ref, w1_ref, b1_ref, w2_ref, b2_ref, out_ref):
    x = x_ref[...]
    h = lax.dot_general(x, w1_ref[...], (((1,), (1,)), ((), ())),
                        preferred_element_type=jnp.float32)
    h = jnp.maximum(h + b1_ref[...], 0.0)
    h = lax.dot_general(h, w2_ref[...], (((1,), (1,)), ((), ())),
                        preferred_element_type=jnp.float32)
    h = h + b2_ref[...]
    norm = jnp.sqrt(jnp.sum(h * h, axis=1, keepdims=True))
    out_ref[...] = h / jnp.maximum(norm, 1e-12)


def _sim_body(rows_ref, emb_ref, out_ref, *, kk):
    rows = rows_ref[...]
    emb = emb_ref[...]
    s = lax.dot_general(rows, emb, (((1,), (1,)), ((), ())),
                        preferred_element_type=jnp.float32)

    def pop(_, tau):
        masked = jnp.where(s < tau, s, -jnp.inf)
        return jnp.max(masked, axis=1, keepdims=True)

    tau0 = jnp.full((s.shape[0], 1), jnp.inf, dtype=jnp.float32)
    tau = lax.fori_loop(0, kk, pop, tau0)
    out_ref[...] = jnp.where(s >= tau, jnp.maximum(s, 0.0), 0.0)


def kernel(features, W1, b1, W2, b2):
    n, d = features.shape
    emb = pl.pallas_call(
        _emb_body,
        out_shape=jax.ShapeDtypeStruct((n, d), jnp.float32),
    )(features, W1, b1.reshape(1, d), W2, b2.reshape(1, d))

    br = 128
    grid = (n // br,)
    out = pl.pallas_call(
        functools.partial(_sim_body, kk=K + 1),
        grid=grid,
        in_specs=[
            pl.BlockSpec((br, d), lambda i: (i, 0)),
            pl.BlockSpec((n, d), lambda i: (0, 0)),
        ],
        out_specs=pl.BlockSpec((br, n), lambda i: (i, 0)),
        out_shape=jax.ShapeDtypeStruct((n, n), jnp.float32),
    )(emb, emb)
    return out
